# Initial kernel scaffold; baseline (speedup 1.0000x reference)
import jax
import jax.numpy as jnp
from jax.experimental import pallas as pl


def kernel(scores, bbox_deltas, im_info):
    raise NotImplementedError("write your pallas kernel here")


# speedup vs baseline: 15.4535x; 15.4535x over previous
import numpy as np
import jax
import jax.numpy as jnp
from jax import lax
from jax.experimental import pallas as pl
from jax.experimental.pallas import tpu as pltpu

_FEAT_STRIDE = 16
_PRE_NMS = 12000
_POST_NMS = 2000
_NMS_THRESH = 0.7
_SUB = 8
_LANE = _PRE_NMS // _SUB


def _base_anchors():
    base = np.array([1.0, 1.0, 16.0, 16.0]) - 1
    w = base[2] - base[0] + 1.0
    h = base[3] - base[1] + 1.0
    xc = base[0] + 0.5 * (w - 1)
    yc = base[1] + 0.5 * (h - 1)
    ratios = np.array([0.5, 1.0, 2.0])
    ws = np.round(np.sqrt(w * h / ratios))
    hs = np.round(ws * ratios)
    rows = []
    scales = np.array([8.0, 16.0, 32.0])
    for i in range(3):
        sw = ws[i] * scales
        sh = hs[i] * scales
        rows.append(
            np.stack(
                [
                    xc - 0.5 * (sw - 1),
                    yc - 0.5 * (sh - 1),
                    xc + 0.5 * (sw - 1),
                    yc + 0.5 * (sh - 1),
                ],
                axis=1,
            )
        )
    return np.concatenate(rows, axis=0).astype(np.float32)


def _grid_anchors(H, W):
    base = _base_anchors()
    sx = (np.arange(W) * _FEAT_STRIDE).astype(np.float32)
    sy = (np.arange(H) * _FEAT_STRIDE).astype(np.float32)
    SX, SY = np.meshgrid(sx, sy)
    shifts = np.stack([SX.ravel(), SY.ravel(), SX.ravel(), SY.ravel()], axis=1)
    return (shifts[:, None, :] + base[None, :, :]).reshape(-1, 4)


def _proposal_nms_kernel(im_ref, a_ref, d_ref, o_ref, s_ref, cnt_ref):
    b = pl.program_id(0)
    h_im = im_ref[b, 0]
    w_im = im_ref[b, 1]
    ax1 = a_ref[0, 0]
    ay1 = a_ref[0, 1]
    ax2 = a_ref[0, 2]
    ay2 = a_ref[0, 3]
    dx = d_ref[0, 0]
    dy = d_ref[0, 1]
    dw = d_ref[0, 2]
    dh = d_ref[0, 3]
    aw = ax2 - ax1 + 1.0
    ah = ay2 - ay1 + 1.0
    cx = ax1 + 0.5 * aw
    cy = ay1 + 0.5 * ah
    pcx = dx * aw + cx
    pcy = dy * ah + cy
    pw = jnp.exp(dw) * aw
    ph = jnp.exp(dh) * ah
    x1 = jnp.clip(pcx - 0.5 * pw, 0.0, w_im - 1.0)
    y1 = jnp.clip(pcy - 0.5 * ph, 0.0, h_im - 1.0)
    x2 = jnp.clip(pcx + 0.5 * pw, 0.0, w_im - 1.0)
    y2 = jnp.clip(pcy + 0.5 * ph, 0.0, h_im - 1.0)
    s_ref[0] = x1
    s_ref[1] = y1
    s_ref[2] = x2
    s_ref[3] = y2
    s_ref[4] = (x2 - x1 + 1.0) * (y2 - y1 + 1.0)
    s_ref[5] = jnp.zeros((_SUB, _LANE), jnp.float32)
    col = lax.broadcasted_iota(jnp.int32, (_POST_NMS, 5), 1)
    o_ref[0] = jnp.where(col == 0, b.astype(jnp.float32), 0.0)
    cnt_ref[0] = 0
    pos = (
        lax.broadcasted_iota(jnp.int32, (_SUB, _LANE), 0) * _LANE
        + lax.broadcasted_iota(jnp.int32, (_SUB, _LANE), 1)
    )

    def cond(st):
        i, c = st
        return jnp.logical_and(i < _PRE_NMS, c < _POST_NMS)

    def body(st):
        i, _ = st
        sel = pos == i

        @pl.when(jnp.sum(jnp.where(sel, s_ref[5], 0.0)) == 0.0)
        def _():
            c = cnt_ref[0]
            x1i = jnp.sum(jnp.where(sel, s_ref[0], 0.0))
            y1i = jnp.sum(jnp.where(sel, s_ref[1], 0.0))
            x2i = jnp.sum(jnp.where(sel, s_ref[2], 0.0))
            y2i = jnp.sum(jnp.where(sel, s_ref[3], 0.0))
            ai = jnp.sum(jnp.where(sel, s_ref[4], 0.0))
            col5 = lax.broadcasted_iota(jnp.int32, (1, 5), 1)
            row = (
                jnp.where(col5 == 0, b.astype(jnp.float32), 0.0)
                + jnp.where(col5 == 1, x1i, 0.0)
                + jnp.where(col5 == 2, y1i, 0.0)
                + jnp.where(col5 == 3, x2i, 0.0)
                + jnp.where(col5 == 4, y2i, 0.0)
            )
            o_ref[0, pl.ds(c, 1), :] = row
            iw = jnp.maximum(
                0.0, jnp.minimum(x2i, s_ref[2]) - jnp.maximum(x1i, s_ref[0]) + 1.0
            )
            ih = jnp.maximum(
                0.0, jnp.minimum(y2i, s_ref[3]) - jnp.maximum(y1i, s_ref[1]) + 1.0
            )
            inter = iw * ih
            ovr = inter / (ai + s_ref[4] - inter)
            sup = jnp.logical_and(pos > i, ovr > _NMS_THRESH)
            s_ref[5] = jnp.where(sup, 1.0, s_ref[5])
            cnt_ref[0] = c + 1

        return i + 1, cnt_ref[0]

    lax.while_loop(cond, body, (jnp.int32(0), jnp.int32(0)))


def kernel(scores, bbox_deltas, im_info):
    B, C, H, W = scores.shape
    A = C // 2
    sc = jnp.transpose(scores[:, A:], (0, 2, 3, 1)).reshape(B, -1)
    deltas = jnp.transpose(bbox_deltas, (0, 2, 3, 1)).reshape(B, -1, 4)
    anchors = jnp.asarray(_grid_anchors(H, W))
    _, idx = lax.top_k(sc, _PRE_NMS)
    a_s = anchors[idx]
    d_s = jnp.take_along_axis(deltas, idx[:, :, None], axis=1)
    a_s = jnp.transpose(a_s, (0, 2, 1)).reshape(B, 4, _SUB, _LANE)
    d_s = jnp.transpose(d_s, (0, 2, 1)).reshape(B, 4, _SUB, _LANE)
    return pl.pallas_call(
        _proposal_nms_kernel,
        grid=(B,),
        in_specs=[
            pl.BlockSpec((B, 3), lambda b: (0, 0)),
            pl.BlockSpec((1, 4, _SUB, _LANE), lambda b: (b, 0, 0, 0)),
            pl.BlockSpec((1, 4, _SUB, _LANE), lambda b: (b, 0, 0, 0)),
        ],
        out_specs=pl.BlockSpec((1, _POST_NMS, 5), lambda b: (b, 0, 0)),
        out_shape=jax.ShapeDtypeStruct((B, _POST_NMS, 5), jnp.float32),
        scratch_shapes=[
            pltpu.VMEM((6, _SUB, _LANE), jnp.float32),
            pltpu.SMEM((1,), jnp.int32),
        ],
    )(im_info, a_s, d_s)


# speedup vs baseline: 62.9736x; 4.0750x over previous
import numpy as np
import jax
import jax.numpy as jnp
from jax import lax
from jax.experimental import pallas as pl
from jax.experimental.pallas import tpu as pltpu

_FEAT_STRIDE = 16
_PRE_NMS = 12000
_POST_NMS = 2000
_NMS_THRESH = 0.7
_SUB = 8
_LANE = _PRE_NMS // _SUB


def _base_anchors():
    base = np.array([1.0, 1.0, 16.0, 16.0]) - 1
    w = base[2] - base[0] + 1.0
    h = base[3] - base[1] + 1.0
    xc = base[0] + 0.5 * (w - 1)
    yc = base[1] + 0.5 * (h - 1)
    ratios = np.array([0.5, 1.0, 2.0])
    ws = np.round(np.sqrt(w * h / ratios))
    hs = np.round(ws * ratios)
    rows = []
    scales = np.array([8.0, 16.0, 32.0])
    for i in range(3):
        sw = ws[i] * scales
        sh = hs[i] * scales
        rows.append(
            np.stack(
                [
                    xc - 0.5 * (sw - 1),
                    yc - 0.5 * (sh - 1),
                    xc + 0.5 * (sw - 1),
                    yc + 0.5 * (sh - 1),
                ],
                axis=1,
            )
        )
    return np.concatenate(rows, axis=0).astype(np.float32)


def _grid_anchors(H, W):
    base = _base_anchors()
    sx = (np.arange(W) * _FEAT_STRIDE).astype(np.float32)
    sy = (np.arange(H) * _FEAT_STRIDE).astype(np.float32)
    SX, SY = np.meshgrid(sx, sy)
    shifts = np.stack([SX.ravel(), SY.ravel(), SX.ravel(), SY.ravel()], axis=1)
    return (shifts[:, None, :] + base[None, :, :]).reshape(-1, 4)


def _proposal_nms_kernel(im_ref, a_ref, d_ref, o_ref, s_ref, cnt_ref):
    b = pl.program_id(0)
    h_im = im_ref[b, 0]
    w_im = im_ref[b, 1]
    ax1 = a_ref[0, 0]
    ay1 = a_ref[0, 1]
    ax2 = a_ref[0, 2]
    ay2 = a_ref[0, 3]
    dx = d_ref[0, 0]
    dy = d_ref[0, 1]
    dw = d_ref[0, 2]
    dh = d_ref[0, 3]
    aw = ax2 - ax1 + 1.0
    ah = ay2 - ay1 + 1.0
    cx = ax1 + 0.5 * aw
    cy = ay1 + 0.5 * ah
    pcx = dx * aw + cx
    pcy = dy * ah + cy
    pw = jnp.exp(dw) * aw
    ph = jnp.exp(dh) * ah
    x1 = jnp.clip(pcx - 0.5 * pw, 0.0, w_im - 1.0)
    y1 = jnp.clip(pcy - 0.5 * ph, 0.0, h_im - 1.0)
    x2 = jnp.clip(pcx + 0.5 * pw, 0.0, w_im - 1.0)
    y2 = jnp.clip(pcy + 0.5 * ph, 0.0, h_im - 1.0)
    s_ref[0] = x1
    s_ref[1] = y1
    s_ref[2] = x2
    s_ref[3] = y2
    s_ref[4] = (x2 - x1 + 1.0) * (y2 - y1 + 1.0)
    col = lax.broadcasted_iota(jnp.int32, (_POST_NMS, 5), 1)
    o_ref[0] = jnp.where(col == 0, b.astype(jnp.float32), 0.0)
    cnt_ref[0] = 0
    posf = (
        lax.broadcasted_iota(jnp.int32, (_SUB, _LANE), 0) * _LANE
        + lax.broadcasted_iota(jnp.int32, (_SUB, _LANE), 1)
    ).astype(jnp.float32)
    s_ref[5] = posf
    pre_f = float(_PRE_NMS)

    def cond(st):
        cur, c = st
        return jnp.logical_and(cur < pre_f, c < _POST_NMS)

    def body(st):
        cur, _ = st
        alive = s_ref[5]
        i = jnp.min(jnp.where(alive >= cur, alive, pre_f))

        @pl.when(i < pre_f)
        def _():
            c = cnt_ref[0]
            sel = s_ref[5] == i
            x1i = jnp.sum(jnp.where(sel, s_ref[0], 0.0))
            y1i = jnp.sum(jnp.where(sel, s_ref[1], 0.0))
            x2i = jnp.sum(jnp.where(sel, s_ref[2], 0.0))
            y2i = jnp.sum(jnp.where(sel, s_ref[3], 0.0))
            ai = jnp.sum(jnp.where(sel, s_ref[4], 0.0))
            col5 = lax.broadcasted_iota(jnp.int32, (1, 5), 1)
            row = (
                jnp.where(col5 == 0, b.astype(jnp.float32), 0.0)
                + jnp.where(col5 == 1, x1i, 0.0)
                + jnp.where(col5 == 2, y1i, 0.0)
                + jnp.where(col5 == 3, x2i, 0.0)
                + jnp.where(col5 == 4, y2i, 0.0)
            )
            o_ref[0, pl.ds(c, 1), :] = row
            iw = jnp.maximum(
                0.0, jnp.minimum(x2i, s_ref[2]) - jnp.maximum(x1i, s_ref[0]) + 1.0
            )
            ih = jnp.maximum(
                0.0, jnp.minimum(y2i, s_ref[3]) - jnp.maximum(y1i, s_ref[1]) + 1.0
            )
            inter = iw * ih
            ovr = inter / (ai + s_ref[4] - inter)
            sup = jnp.logical_and(s_ref[5] > i, ovr > _NMS_THRESH)
            s_ref[5] = jnp.where(sup, pre_f, s_ref[5])
            cnt_ref[0] = c + 1

        return i + 1.0, cnt_ref[0]

    lax.while_loop(cond, body, (jnp.float32(0.0), jnp.int32(0)))


def kernel(scores, bbox_deltas, im_info):
    B, C, H, W = scores.shape
    A = C // 2
    sc = jnp.transpose(scores[:, A:], (0, 2, 3, 1)).reshape(B, -1)
    deltas = jnp.transpose(bbox_deltas, (0, 2, 3, 1)).reshape(B, -1, 4)
    anchors = jnp.asarray(_grid_anchors(H, W))
    _, idx = lax.top_k(sc, _PRE_NMS)
    a_s = anchors[idx]
    d_s = jnp.take_along_axis(deltas, idx[:, :, None], axis=1)
    a_s = jnp.transpose(a_s, (0, 2, 1)).reshape(B, 4, _SUB, _LANE)
    d_s = jnp.transpose(d_s, (0, 2, 1)).reshape(B, 4, _SUB, _LANE)
    return pl.pallas_call(
        _proposal_nms_kernel,
        grid=(B,),
        in_specs=[
            pl.BlockSpec((B, 3), lambda b: (0, 0)),
            pl.BlockSpec((1, 4, _SUB, _LANE), lambda b: (b, 0, 0, 0)),
            pl.BlockSpec((1, 4, _SUB, _LANE), lambda b: (b, 0, 0, 0)),
        ],
        out_specs=pl.BlockSpec((1, _POST_NMS, 5), lambda b: (b, 0, 0)),
        out_shape=jax.ShapeDtypeStruct((B, _POST_NMS, 5), jnp.float32),
        scratch_shapes=[
            pltpu.VMEM((6, _SUB, _LANE), jnp.float32),
            pltpu.SMEM((1,), jnp.int32),
        ],
    )(im_info, a_s, d_s)
